# baseline (device time: 47539 ns/iter reference)
import jax
import jax.numpy as jnp
from jax import lax
from jax.experimental import pallas as pl
from jax.experimental.pallas import tpu as pltpu

N_DEV = 4
CAPACITY = 204

_signal = getattr(pl, "semaphore_signal", None) or pltpu.semaphore_signal
_sem_wait = getattr(pl, "semaphore_wait", None) or pltpu.semaphore_wait
_CompilerParams = getattr(pltpu, "CompilerParams", None) or pltpu.TPUCompilerParams


def kernel(x, router_W, route_idx, expert_W):
    del router_W
    n_tok, d_model = x.shape
    e_loc, _, d_ff = expert_W.shape
    n_exp = N_DEV * e_loc

    def body(x_ref, idx_ref, w_ref, out_ref, w_slots, route_slots,
             w_send_sems, w_recv_sems, r_send_sems, r_recv_sems):
        my = lax.axis_index("i")

        w_slots[0] = w_ref[...].astype(jnp.bfloat16)
        route_slots[0] = idx_ref[...]

        barrier = pltpu.get_barrier_semaphore()
        for k in range(1, N_DEV):
            _signal(
                barrier, inc=1,
                device_id=(lax.rem(my + k, N_DEV),),
                device_id_type=pl.DeviceIdType.MESH,
            )
        _sem_wait(barrier, N_DEV - 1)

        sends = []
        for k in range(1, N_DEV):
            dst = (lax.rem(my + k, N_DEV),)
            rr = pltpu.make_async_remote_copy(
                src_ref=route_slots.at[0],
                dst_ref=route_slots.at[k],
                send_sem=r_send_sems.at[k - 1],
                recv_sem=r_recv_sems.at[k - 1],
                device_id=dst,
                device_id_type=pl.DeviceIdType.MESH,
            )
            rw = pltpu.make_async_remote_copy(
                src_ref=w_slots.at[0],
                dst_ref=w_slots.at[k],
                send_sem=w_send_sems.at[k - 1],
                recv_sem=w_recv_sems.at[k - 1],
                device_id=dst,
                device_id_type=pl.DeviceIdType.MESH,
            )
            rr.start()
            rw.start()
            sends.append((rr, rw))

        for k in range(1, N_DEV):
            pltpu.make_async_remote_copy(
                src_ref=route_slots.at[0],
                dst_ref=route_slots.at[k],
                send_sem=r_send_sems.at[k - 1],
                recv_sem=r_recv_sems.at[k - 1],
                device_id=(my,),
                device_id_type=pl.DeviceIdType.MESH,
            ).wait_recv()

        cols = lax.broadcasted_iota(jnp.int32, (1, n_exp), 1)
        s_col = cols // e_loc
        j_col = cols % e_loc
        e_map = lax.rem(my - s_col + N_DEV, N_DEV) * e_loc + j_col

        oh_mine = (route_slots[0] == e_map).astype(jnp.float32)
        prev = jnp.zeros((1, n_exp), jnp.float32)
        for s in range(1, N_DEV):
            oh_s = (route_slots[s] == e_map).astype(jnp.float32)
            cnt_s = jnp.sum(oh_s, axis=0, keepdims=True)
            prev = prev + jnp.where(my >= s, cnt_s, 0.0)

        ti = lax.broadcasted_iota(jnp.int32, (n_tok, n_tok), 0)
        tj = lax.broadcasted_iota(jnp.int32, (n_tok, n_tok), 1)
        tri = (tj < ti).astype(jnp.bfloat16)
        rank = jnp.dot(tri, oh_mine.astype(jnp.bfloat16),
                       preferred_element_type=jnp.float32)

        keep = oh_mine * (rank + prev < CAPACITY).astype(jnp.float32)
        keep_bf = keep.astype(jnp.bfloat16)
        x_bf = x_ref[...].astype(jnp.bfloat16)

        acc = jnp.zeros((n_tok, d_ff), jnp.float32)
        for s in range(N_DEV):
            if s > 0:
                pltpu.make_async_remote_copy(
                    src_ref=w_slots.at[0],
                    dst_ref=w_slots.at[s],
                    send_sem=w_send_sems.at[s - 1],
                    recv_sem=w_recv_sems.at[s - 1],
                    device_id=(my,),
                    device_id_type=pl.DeviceIdType.MESH,
                ).wait_recv()
            for j in range(e_loc):
                c = s * e_loc + j
                xm = x_bf * keep_bf[:, c:c + 1]
                acc = acc + jnp.dot(xm, w_slots[s, j],
                                    preferred_element_type=jnp.float32)

        for rr, rw in sends:
            rr.wait_send()
            rw.wait_send()

        out_ref[...] = acc

    return pl.pallas_call(
        body,
        out_shape=jax.ShapeDtypeStruct((n_tok, d_ff), jnp.float32),
        in_specs=[
            pl.BlockSpec(memory_space=pltpu.VMEM),
            pl.BlockSpec(memory_space=pltpu.VMEM),
            pl.BlockSpec(memory_space=pltpu.VMEM),
        ],
        out_specs=pl.BlockSpec(memory_space=pltpu.VMEM),
        scratch_shapes=[
            pltpu.VMEM((N_DEV, e_loc, d_model, d_ff), jnp.bfloat16),
            pltpu.VMEM((N_DEV, n_tok, 1), jnp.int32),
            pltpu.SemaphoreType.DMA((N_DEV - 1,)),
            pltpu.SemaphoreType.DMA((N_DEV - 1,)),
            pltpu.SemaphoreType.DMA((N_DEV - 1,)),
            pltpu.SemaphoreType.DMA((N_DEV - 1,)),
        ],
        compiler_params=_CompilerParams(collective_id=0),
    )(x, route_idx, expert_W)


# device time: 36756 ns/iter; 1.2934x vs baseline; 1.2934x over previous
import jax
import jax.numpy as jnp
from jax import lax
from jax.experimental import pallas as pl
from jax.experimental.pallas import tpu as pltpu

N_DEV = 4
CAPACITY = 204

_signal = getattr(pl, "semaphore_signal", None) or pltpu.semaphore_signal
_sem_wait = getattr(pl, "semaphore_wait", None) or pltpu.semaphore_wait
_CompilerParams = getattr(pltpu, "CompilerParams", None) or pltpu.TPUCompilerParams


def kernel(x, router_W, route_idx, expert_W):
    del router_W
    n_tok, d_model = x.shape
    e_loc, _, d_ff = expert_W.shape
    n_exp = N_DEV * e_loc

    def body(x_ref, idx_ref, w_ref, out_ref, w_slots, cnt_out, cnt_in,
             w_send_sems, w_recv_sems, c_send_sems, c_recv_sems):
        my = lax.axis_index("i")

        def e_map_for(pos):
            cols = lax.broadcasted_iota(jnp.int32, (1, n_exp), 1)
            origin = lax.rem(pos - cols // e_loc + N_DEV, N_DEV)
            return origin * e_loc + cols % e_loc

        w_slots[0] = w_ref[...].astype(jnp.bfloat16)
        for k in range(1, N_DEV):
            oh_k = (idx_ref[...] == e_map_for(lax.rem(my + k, N_DEV)))
            cnt_out[k] = jnp.sum(oh_k.astype(jnp.float32), axis=0,
                                 keepdims=True)

        barrier = pltpu.get_barrier_semaphore()
        for k in range(1, N_DEV):
            _signal(
                barrier, inc=1,
                device_id=(lax.rem(my + k, N_DEV),),
                device_id_type=pl.DeviceIdType.MESH,
            )
        _sem_wait(barrier, N_DEV - 1)

        sends = []
        for k in range(1, N_DEV):
            dst = (lax.rem(my + k, N_DEV),)
            rc = pltpu.make_async_remote_copy(
                src_ref=cnt_out.at[k],
                dst_ref=cnt_in.at[k],
                send_sem=c_send_sems.at[k - 1],
                recv_sem=c_recv_sems.at[k - 1],
                device_id=dst,
                device_id_type=pl.DeviceIdType.MESH,
            )
            rw = pltpu.make_async_remote_copy(
                src_ref=w_slots.at[0],
                dst_ref=w_slots.at[k],
                send_sem=w_send_sems.at[k - 1],
                recv_sem=w_recv_sems.at[k - 1],
                device_id=dst,
                device_id_type=pl.DeviceIdType.MESH,
            )
            rc.start()
            rw.start()
            sends.append(rc)
            sends.append(rw)

        oh_mine = (idx_ref[...] == e_map_for(my)).astype(jnp.float32)
        ti = lax.broadcasted_iota(jnp.int32, (n_tok, n_tok), 0)
        tj = lax.broadcasted_iota(jnp.int32, (n_tok, n_tok), 1)
        tri = (tj < ti).astype(jnp.bfloat16)
        rank = jnp.dot(tri, oh_mine.astype(jnp.bfloat16),
                       preferred_element_type=jnp.float32)
        x_bf = x_ref[...].astype(jnp.bfloat16)

        for k in range(1, N_DEV):
            pltpu.make_async_remote_copy(
                src_ref=cnt_out.at[k],
                dst_ref=cnt_in.at[k],
                send_sem=c_send_sems.at[k - 1],
                recv_sem=c_recv_sems.at[k - 1],
                device_id=(my,),
                device_id_type=pl.DeviceIdType.MESH,
            ).wait_recv()
        prev = jnp.zeros((1, n_exp), jnp.float32)
        for s in range(1, N_DEV):
            prev = prev + jnp.where(my >= s, cnt_in[s], 0.0)

        keep = oh_mine * (rank + prev < CAPACITY).astype(jnp.float32)
        keep_bf = keep.astype(jnp.bfloat16)

        acc = jnp.zeros((n_tok, d_ff), jnp.float32)
        for s in (0, 1, 2, 3):
            if s > 0:
                pltpu.make_async_remote_copy(
                    src_ref=w_slots.at[0],
                    dst_ref=w_slots.at[s],
                    send_sem=w_send_sems.at[s - 1],
                    recv_sem=w_recv_sems.at[s - 1],
                    device_id=(my,),
                    device_id_type=pl.DeviceIdType.MESH,
                ).wait_recv()
            for j in range(e_loc):
                c = s * e_loc + j
                xm = x_bf * keep_bf[:, c:c + 1]
                acc = acc + jnp.dot(xm, w_slots[s, j],
                                    preferred_element_type=jnp.float32)

        for rdma in sends:
            rdma.wait_send()

        out_ref[...] = acc

    return pl.pallas_call(
        body,
        out_shape=jax.ShapeDtypeStruct((n_tok, d_ff), jnp.float32),
        in_specs=[
            pl.BlockSpec(memory_space=pltpu.VMEM),
            pl.BlockSpec(memory_space=pltpu.VMEM),
            pl.BlockSpec(memory_space=pltpu.VMEM),
        ],
        out_specs=pl.BlockSpec(memory_space=pltpu.VMEM),
        scratch_shapes=[
            pltpu.VMEM((N_DEV, e_loc, d_model, d_ff), jnp.bfloat16),
            pltpu.VMEM((N_DEV, 1, n_exp), jnp.float32),
            pltpu.VMEM((N_DEV, 1, n_exp), jnp.float32),
            pltpu.SemaphoreType.DMA((N_DEV - 1,)),
            pltpu.SemaphoreType.DMA((N_DEV - 1,)),
            pltpu.SemaphoreType.DMA((N_DEV - 1,)),
            pltpu.SemaphoreType.DMA((N_DEV - 1,)),
        ],
        compiler_params=_CompilerParams(collective_id=0),
    )(x, route_idx, expert_W)
